# baseline (speedup 1.0000x reference)
import functools

import jax
import jax.numpy as jnp
from jax.experimental import pallas as pl
from jax.experimental.pallas import tpu as pltpu

_LN_EPS = 1e-5
_VMEM_LIMIT = 64 * 1024 * 1024


def _conv_ln_body(mask_ref, xp_ref, w_ref, b_ref, g_ref, be_ref, out_ref, y_ref, *,
                  bsz, ac, mc, wc, ho, wo, cout, cin2, ck, nck, tg):
    rows = ho * wc
    inv_cnt = 1.0 / float(cout * ho * wo)
    taps = [(p, a, m) for p in range(2) for a in range(ac) for m in range(mc)]
    ntap = len(taps)
    bias = b_ref[...]
    for s in range(bsz):
        s1 = jnp.zeros((cout, ck), jnp.float32)
        s2 = jnp.zeros((cout, ck), jnp.float32)
        for k in range(nck):
            base = k * ck
            yc = None
            for g0 in range(0, ntap, tg):
                grp = taps[g0:g0 + tg]
                rhs = jnp.concatenate(
                    [xp_ref[s, p, :, pl.ds(a * wc + m + base, ck)] for (p, a, m) in grp],
                    axis=0)
                d = jnp.dot(w_ref[:, g0 * cin2:(g0 + len(grp)) * cin2], rhs,
                            preferred_element_type=jnp.float32)
                yc = d if yc is None else yc + d
            yc = jnp.maximum(yc + bias, 0.0)
            y_ref[:, base:base + ck] = yc
            t = yc * mask_ref[:, base:base + ck]
            s1 = s1 + t
            s2 = s2 + t * yc
        mu = jnp.sum(s1, keepdims=True) * inv_cnt
        ex2 = jnp.sum(s2, keepdims=True) * inv_cnt
        rstd = jax.lax.rsqrt(ex2 - mu * mu + _LN_EPS)
        y = y_ref[:, :rows]
        rg = g_ref[...].astype(jnp.float32) * rstd
        out_ref[s] = ((y - mu) * rg
                      + be_ref[...].astype(jnp.float32)).astype(out_ref.dtype)


def _final_body(mask_ref, xp_ref, w_ref, b_ref, g_ref, be_ref, wl_ref, bl_ref,
                out_ref, pool_ref, *, bsz, tg):
    ac, mc, wc, ho, wo, cout, cin2, ck = 2, 2, 7, 6, 6, 128, 128, 128
    inv_cnt = 1.0 / float(cout * ho * wo)
    inv_pool = 1.0 / float(ho * wo)
    taps = [(p, a, m) for p in range(2) for a in range(ac) for m in range(mc)]
    mask = mask_ref[...]
    bias = b_ref[...]
    for s in range(bsz):
        yc = None
        for g0 in range(0, len(taps), tg):
            grp = taps[g0:g0 + tg]
            rhs = jnp.concatenate(
                [xp_ref[s, p, :, pl.ds(a * wc + m, ck)] for (p, a, m) in grp],
                axis=0)
            d = jnp.dot(w_ref[:, g0 * cin2:(g0 + len(grp)) * cin2], rhs,
                        preferred_element_type=jnp.float32)
            yc = d if yc is None else yc + d
        yc = jnp.maximum(yc + bias, 0.0)
        t = yc * mask
        mu = jnp.sum(t, keepdims=True) * inv_cnt
        ex2 = jnp.sum(t * yc, keepdims=True) * inv_cnt
        rstd = jax.lax.rsqrt(ex2 - mu * mu + _LN_EPS)
        rg = g_ref[...].astype(jnp.float32) * rstd
        yn = (yc[:, :ho * wc] - mu) * rg + be_ref[...].astype(jnp.float32)
        pool_ref[:, s:s + 1] = jnp.sum(yn * mask[:, :ho * wc], axis=1,
                                       keepdims=True) * inv_pool
    out_ref[0] = (jnp.dot(wl_ref[...], pool_ref[...],
                          preferred_element_type=jnp.float32) + bl_ref[...])


def _conv_stage(xp, mask, w2d, bias, gamma, beta, *, ac, mc, wc, ho, wo, cout,
                cin2, bsz, ck, nck, tg):
    n, _, _, r = xp.shape
    rows = ho * wc
    kp = 2 * ac * mc * cin2
    body = functools.partial(_conv_ln_body, bsz=bsz, ac=ac, mc=mc, wc=wc, ho=ho,
                             wo=wo, cout=cout, cin2=cin2, ck=ck, nck=nck, tg=tg)
    flops = 2 * n * cout * kp * rows
    bytes_acc = n * (2 * cin2 * r + cout * rows) * 2
    return pl.pallas_call(
        body,
        out_shape=jax.ShapeDtypeStruct((n, cout, rows), jnp.bfloat16),
        grid=(n // bsz,),
        in_specs=[
            pl.BlockSpec((1, nck * ck), lambda i: (0, 0)),
            pl.BlockSpec((bsz, 2, cin2, r), lambda i: (i, 0, 0, 0)),
            pl.BlockSpec((cout, kp), lambda i: (0, 0)),
            pl.BlockSpec((cout, 1), lambda i: (0, 0)),
            pl.BlockSpec((cout, rows), lambda i: (0, 0)),
            pl.BlockSpec((cout, rows), lambda i: (0, 0)),
        ],
        out_specs=pl.BlockSpec((bsz, cout, rows), lambda i: (i, 0, 0)),
        scratch_shapes=[pltpu.VMEM((cout, nck * ck), jnp.float32)],
        compiler_params=pltpu.CompilerParams(
            dimension_semantics=("parallel",), vmem_limit_bytes=_VMEM_LIMIT),
        cost_estimate=pl.CostEstimate(flops=int(flops), transcendentals=int(n),
                                      bytes_accessed=int(bytes_acc)),
    )(mask, xp, w2d, bias, gamma, beta)


def _final_stage(xp, mask, w2d, bias, gamma, beta, wl, bl, *, bsz, tg):
    n, _, _, r = xp.shape
    kp = 8 * 128
    flops = 2 * n * (128 * kp * 42 + 27 * 128)
    bytes_acc = n * (2 * 128 * r + 27) * 2
    return pl.pallas_call(
        functools.partial(_final_body, bsz=bsz, tg=tg),
        out_shape=jax.ShapeDtypeStruct((n // bsz, 27, bsz), jnp.float32),
        grid=(n // bsz,),
        in_specs=[
            pl.BlockSpec((1, 128), lambda i: (0, 0)),
            pl.BlockSpec((bsz, 2, 128, r), lambda i: (i, 0, 0, 0)),
            pl.BlockSpec((128, kp), lambda i: (0, 0)),
            pl.BlockSpec((128, 1), lambda i: (0, 0)),
            pl.BlockSpec((128, 42), lambda i: (0, 0)),
            pl.BlockSpec((128, 42), lambda i: (0, 0)),
            pl.BlockSpec((27, 128), lambda i: (0, 0)),
            pl.BlockSpec((27, 1), lambda i: (0, 0)),
        ],
        out_specs=pl.BlockSpec((1, 27, bsz), lambda i: (i, 0, 0)),
        scratch_shapes=[pltpu.VMEM((128, bsz), jnp.float32)],
        compiler_params=pltpu.CompilerParams(
            dimension_semantics=("parallel",), vmem_limit_bytes=_VMEM_LIMIT),
        cost_estimate=pl.CostEstimate(flops=int(flops), transcendentals=int(n),
                                      bytes_accessed=int(bytes_acc)),
    )(mask, xp, w2d, bias, gamma, beta, wl, bl)


def _parity_restack(y, c, ho, wwide, wo, r_pad):
    n = y.shape[0]
    y = y.reshape(n, c, ho, wwide)[:, :, :, :wo]
    hp, wp = ho + (ho & 1), wo + (wo & 1)
    if (hp, wp) != (ho, wo):
        y = jnp.pad(y, ((0, 0), (0, 0), (0, hp - ho), (0, wp - wo)))
    hc, wc = hp // 2, wp // 2
    t = y.reshape(n, c, hc, 2, wc, 2)
    t = jnp.transpose(t, (0, 3, 5, 1, 2, 4)).reshape(n, 2, 2 * c, hc * wc)
    return jnp.pad(t, ((0, 0), (0, 0), (0, 0), (0, r_pad - hc * wc)))


def _pad_lanes(m, lanes):
    return jnp.pad(m, ((0, 0), (0, lanes - m.shape[1])))


def kernel(x, w1, b1, g1, be1, m1, w2, b2, g2, be2, m2, w3, b3, g3, be3, m3,
           w4, b4, g4, be4, m4, wl, bl):
    n = x.shape[0]
    bsz = 8 if n >= 16 else (4 if n >= 8 else (2 if n >= 4 else 1))
    n_pad = ((n + bsz - 1) // bsz) * bsz
    if n_pad > n:
        x = jnp.pad(x, ((0, n_pad - n), (0, 0), (0, 0), (0, 0)))

    xb = jnp.pad(x.astype(jnp.bfloat16), ((0, 0), (0, 0), (0, 0), (0, 1)))
    xp1 = jnp.transpose(xb.reshape(n_pad, 60, 2, 60, 2, 8),
                        (0, 2, 4, 5, 1, 3)).reshape(n_pad, 2, 16, 3600)
    xp1 = jnp.pad(xp1, ((0, 0), (0, 0), (0, 0), (0, 112)))

    y1 = _conv_stage(xp1, _pad_lanes(m1, 3584), w1, b1, g1, be1,
                     ac=3, mc=3, wc=60, ho=58, wo=58, cout=16, cin2=16,
                     bsz=bsz, ck=256, nck=14, tg=9)

    xp2 = _parity_restack(y1, 16, 58, 60, 58, 1152)
    y2 = _conv_stage(xp2, _pad_lanes(m2, 1024), w2, b2, g2, be2,
                     ac=2, mc=2, wc=29, ho=28, wo=28, cout=32, cin2=32,
                     bsz=bsz, ck=256, nck=4, tg=8)

    xp3 = _parity_restack(y2, 32, 28, 29, 28, 384)
    y3 = _conv_stage(xp3, _pad_lanes(m3, 256), w3, b3, g3, be3,
                     ac=2, mc=2, wc=14, ho=13, wo=13, cout=64, cin2=64,
                     bsz=bsz, ck=256, nck=1, tg=4)

    xp4 = _parity_restack(y3, 64, 13, 14, 13, 256)
    out = _final_stage(xp4, _pad_lanes(m4, 128), w4, b4, g4, be4, wl, bl,
                       bsz=bsz, tg=4)
    out = jnp.transpose(out, (0, 2, 1)).reshape(n_pad, 27)
    return out[:n]


# speedup vs baseline: 10.2098x; 10.2098x over previous
import functools

import jax
import jax.numpy as jnp
from jax.experimental import pallas as pl
from jax.experimental.pallas import tpu as pltpu

_LN_EPS = 1e-5
_VMEM_LIMIT = 64 * 1024 * 1024


def _conv_ln_body(mask_ref, xp_ref, w_ref, b_ref, g_ref, be_ref, out_ref, y_ref, *,
                  bsz, ac, mc, wc, ho, wo, cout, cin2, ck, nck, tg):
    rows = ho * wc
    inv_cnt = 1.0 / float(cout * ho * wo)
    taps = [(p, a, m) for p in range(2) for a in range(ac) for m in range(mc)]
    ntap = len(taps)
    bias = b_ref[...]
    for s in range(bsz):
        s1 = jnp.zeros((cout, ck), jnp.float32)
        s2 = jnp.zeros((cout, ck), jnp.float32)
        for k in range(nck):
            base = k * ck
            yc = None
            for g0 in range(0, ntap, tg):
                grp = taps[g0:g0 + tg]
                rhs = jnp.concatenate(
                    [xp_ref[s, p, :, pl.ds(a * wc + m + base, ck)] for (p, a, m) in grp],
                    axis=0)
                d = jnp.dot(w_ref[:, g0 * cin2:(g0 + len(grp)) * cin2], rhs,
                            preferred_element_type=jnp.float32)
                yc = d if yc is None else yc + d
            yc = jnp.maximum(yc + bias, 0.0)
            y_ref[:, base:base + ck] = yc
            t = yc * mask_ref[:, base:base + ck]
            s1 = s1 + t
            s2 = s2 + t * yc
        mu = jnp.sum(s1, keepdims=True) * inv_cnt
        ex2 = jnp.sum(s2, keepdims=True) * inv_cnt
        rstd = jax.lax.rsqrt(ex2 - mu * mu + _LN_EPS)
        y = y_ref[:, :rows]
        rg = g_ref[...].astype(jnp.float32) * rstd
        out_ref[s] = ((y - mu) * rg
                      + be_ref[...].astype(jnp.float32)).astype(out_ref.dtype)


def _final_body(mask_ref, xp_ref, w_ref, b_ref, g_ref, be_ref, wl_ref, bl_ref,
                out_ref, pool_ref, *, bsz, tg):
    ac, mc, wc, ho, wo, cout, cin2, ck = 2, 2, 7, 6, 6, 128, 128, 128
    inv_cnt = 1.0 / float(cout * ho * wo)
    inv_pool = 1.0 / float(ho * wo)
    taps = [(p, a, m) for p in range(2) for a in range(ac) for m in range(mc)]
    mask = mask_ref[...]
    bias = b_ref[...]
    for s in range(bsz):
        yc = None
        for g0 in range(0, len(taps), tg):
            grp = taps[g0:g0 + tg]
            rhs = jnp.concatenate(
                [xp_ref[s, p, :, pl.ds(a * wc + m, ck)] for (p, a, m) in grp],
                axis=0)
            d = jnp.dot(w_ref[:, g0 * cin2:(g0 + len(grp)) * cin2], rhs,
                        preferred_element_type=jnp.float32)
            yc = d if yc is None else yc + d
        yc = jnp.maximum(yc + bias, 0.0)
        t = yc * mask
        mu = jnp.sum(t, keepdims=True) * inv_cnt
        ex2 = jnp.sum(t * yc, keepdims=True) * inv_cnt
        rstd = jax.lax.rsqrt(ex2 - mu * mu + _LN_EPS)
        rg = g_ref[...].astype(jnp.float32) * rstd
        yn = (yc[:, :ho * wc] - mu) * rg + be_ref[...].astype(jnp.float32)
        pool_ref[:, s:s + 1] = jnp.sum(yn * mask[:, :ho * wc], axis=1,
                                       keepdims=True) * inv_pool
    res = (jnp.dot(wl_ref[...], pool_ref[...],
                   preferred_element_type=jnp.float32) + bl_ref[...])
    out_ref[...] = jnp.transpose(res, (1, 0))


def _conv_stage(xp, mask, w2d, bias, gamma, beta, *, ac, mc, wc, ho, wo, cout,
                cin2, bsz, ck, nck, tg):
    n, _, _, r = xp.shape
    rows = ho * wc
    kp = 2 * ac * mc * cin2
    body = functools.partial(_conv_ln_body, bsz=bsz, ac=ac, mc=mc, wc=wc, ho=ho,
                             wo=wo, cout=cout, cin2=cin2, ck=ck, nck=nck, tg=tg)
    flops = 2 * n * cout * kp * rows
    bytes_acc = n * (2 * cin2 * r + cout * rows) * 2
    return pl.pallas_call(
        body,
        out_shape=jax.ShapeDtypeStruct((n, cout, rows), jnp.bfloat16),
        grid=(n // bsz,),
        in_specs=[
            pl.BlockSpec((1, nck * ck), lambda i: (0, 0)),
            pl.BlockSpec((bsz, 2, cin2, r), lambda i: (i, 0, 0, 0)),
            pl.BlockSpec((cout, kp), lambda i: (0, 0)),
            pl.BlockSpec((cout, 1), lambda i: (0, 0)),
            pl.BlockSpec((cout, rows), lambda i: (0, 0)),
            pl.BlockSpec((cout, rows), lambda i: (0, 0)),
        ],
        out_specs=pl.BlockSpec((bsz, cout, rows), lambda i: (i, 0, 0)),
        scratch_shapes=[pltpu.VMEM((cout, nck * ck), jnp.float32)],
        compiler_params=pltpu.CompilerParams(
            dimension_semantics=("parallel",), vmem_limit_bytes=_VMEM_LIMIT),
        cost_estimate=pl.CostEstimate(flops=int(flops), transcendentals=int(n),
                                      bytes_accessed=int(bytes_acc)),
    )(mask, xp, w2d, bias, gamma, beta)


def _final_stage(xp, mask, w2d, bias, gamma, beta, wl, bl, *, bsz, tg):
    n, _, _, r = xp.shape
    kp = 8 * 128
    flops = 2 * n * (128 * kp * 42 + 27 * 128)
    bytes_acc = n * (2 * 128 * r + 27) * 2
    return pl.pallas_call(
        functools.partial(_final_body, bsz=bsz, tg=tg),
        out_shape=jax.ShapeDtypeStruct((n, 27), jnp.float32),
        grid=(n // bsz,),
        in_specs=[
            pl.BlockSpec((1, 128), lambda i: (0, 0)),
            pl.BlockSpec((bsz, 2, 128, r), lambda i: (i, 0, 0, 0)),
            pl.BlockSpec((128, kp), lambda i: (0, 0)),
            pl.BlockSpec((128, 1), lambda i: (0, 0)),
            pl.BlockSpec((128, 42), lambda i: (0, 0)),
            pl.BlockSpec((128, 42), lambda i: (0, 0)),
            pl.BlockSpec((27, 128), lambda i: (0, 0)),
            pl.BlockSpec((27, 1), lambda i: (0, 0)),
        ],
        out_specs=pl.BlockSpec((bsz, 27), lambda i: (i, 0)),
        scratch_shapes=[pltpu.VMEM((128, bsz), jnp.float32)],
        compiler_params=pltpu.CompilerParams(
            dimension_semantics=("parallel",), vmem_limit_bytes=_VMEM_LIMIT),
        cost_estimate=pl.CostEstimate(flops=int(flops), transcendentals=int(n),
                                      bytes_accessed=int(bytes_acc)),
    )(mask, xp, w2d, bias, gamma, beta, wl, bl)


def _deinterleave_idx(shape, dim, npair):
    j = jax.lax.broadcasted_iota(jnp.int32, shape, dim)
    return jnp.where(j < npair, 2 * j, 2 * (j - npair) + 1)


def _input_restack_body(x_ref, out_ref, *, bsz):
    out_ref[...] = jnp.zeros(out_ref.shape, out_ref.dtype)
    hidx = _deinterleave_idx((840, 120), 1, 60)
    for s in range(bsz):
        t = jnp.transpose(x_ref[s], (1, 0))
        g = jnp.take_along_axis(t, hidx, axis=1)
        g = g.reshape(60, 14, 120)
        for p in range(2):
            for q in range(2):
                for c in range(7):
                    v = g[:, 7 * q + c, 60 * p:60 * (p + 1)]
                    out_ref[s, p, q * 8 + c, :60, :] = (
                        jnp.transpose(v, (1, 0)).astype(out_ref.dtype))


def _restack_body(y_ref, out_ref, *, bsz, cout, ho, wwide, wo, hc, wc):
    out_ref[...] = jnp.zeros(out_ref.shape, out_ref.dtype)
    widx = _deinterleave_idx((cout, ho // 2, 2 * wc), 2, wc)
    for s in range(bsz):
        for p in range(2):
            a = y_ref[s, :, :, p, :].astype(jnp.float32)
            g = jnp.take_along_axis(a, widx, axis=2).astype(out_ref.dtype)
            for q in range(2):
                wv = min((wo - q + 1) // 2, wc)
                out_ref[s, p, q * cout:(q + 1) * cout, :ho // 2, :wv] = (
                    g[:, :, q * wc:q * wc + wv])


def _restack_odd_body(y_ref, out_ref, *, bsz, cout, ho, wwide, wo, hc, wc):
    out_ref[...] = jnp.zeros(out_ref.shape, out_ref.dtype)
    widx = _deinterleave_idx((cout, ho, 2 * wc), 2, wc)
    for s in range(bsz):
        a = y_ref[s].astype(jnp.float32)
        g = jnp.take_along_axis(a, widx, axis=2).astype(out_ref.dtype)
        ge = g[:, :ho - 1, :].reshape(cout, ho // 2, 2, 2 * wc)
        for p in range(2):
            hv = (ho - p + 1) // 2
            for q in range(2):
                wv = min((wo - q + 1) // 2, wc)
                out_ref[s, p, q * cout:(q + 1) * cout, :ho // 2, :wv] = (
                    ge[:, :, p, q * wc:q * wc + wv])
                if p == 0 and hv > ho // 2:
                    out_ref[s, p, q * cout:(q + 1) * cout,
                            ho // 2:hv, :wv] = g[:, ho - 1:ho, q * wc:q * wc + wv]


def _input_restack(x, *, bsz):
    n = x.shape[0]
    return pl.pallas_call(
        functools.partial(_input_restack_body, bsz=bsz),
        out_shape=jax.ShapeDtypeStruct((n, 2, 16, 62, 60), jnp.bfloat16),
        grid=(n // bsz,),
        in_specs=[pl.BlockSpec((bsz, 120, 840), lambda i: (i, 0, 0))],
        out_specs=pl.BlockSpec((bsz, 2, 16, 62, 60), lambda i: (i, 0, 0, 0, 0)),
        compiler_params=pltpu.CompilerParams(
            dimension_semantics=("parallel",), vmem_limit_bytes=_VMEM_LIMIT),
    )(x)


def _restack(y, *, bsz, cout, ho, wwide, wo, hcp):
    n = y.shape[0]
    hc, wc = (ho + 1) // 2, (wo + 1) // 2
    kw = dict(bsz=bsz, cout=cout, ho=ho, wwide=wwide, wo=wo, hc=hc, wc=wc)
    if ho % 2 == 0:
        y = y.reshape(n, cout, ho // 2, 2, wwide)
        body = functools.partial(_restack_body, **kw)
        in_spec = pl.BlockSpec((bsz, cout, ho // 2, 2, wwide),
                               lambda i: (i, 0, 0, 0, 0))
    else:
        y = y.reshape(n, cout, ho, wwide)
        body = functools.partial(_restack_odd_body, **kw)
        in_spec = pl.BlockSpec((bsz, cout, ho, wwide), lambda i: (i, 0, 0, 0))
    out = pl.pallas_call(
        body,
        out_shape=jax.ShapeDtypeStruct((n, 2, 2 * cout, hcp, wc), jnp.bfloat16),
        grid=(n // bsz,),
        in_specs=[in_spec],
        out_specs=pl.BlockSpec((bsz, 2, 2 * cout, hcp, wc),
                               lambda i: (i, 0, 0, 0, 0)),
        compiler_params=pltpu.CompilerParams(
            dimension_semantics=("parallel",), vmem_limit_bytes=_VMEM_LIMIT),
    )(y)
    return out.reshape(n, 2, 2 * cout, hcp * wc)


def _pad_lanes(m, lanes):
    return jnp.pad(m, ((0, 0), (0, lanes - m.shape[1])))


def kernel(x, w1, b1, g1, be1, m1, w2, b2, g2, be2, m2, w3, b3, g3, be3, m3,
           w4, b4, g4, be4, m4, wl, bl):
    n = x.shape[0]
    bsz = 8 if n >= 16 else (4 if n >= 8 else (2 if n >= 4 else 1))
    n_pad = ((n + bsz - 1) // bsz) * bsz
    if n_pad > n:
        x = jnp.pad(x, ((0, n_pad - n), (0, 0), (0, 0), (0, 0)))

    xp1 = _input_restack(x.reshape(n_pad, 120, 840), bsz=bsz)
    xp1 = xp1.reshape(n_pad, 2, 16, 62 * 60)

    y1 = _conv_stage(xp1, _pad_lanes(m1, 3584), w1, b1, g1, be1,
                     ac=3, mc=3, wc=60, ho=58, wo=58, cout=16, cin2=16,
                     bsz=bsz, ck=256, nck=14, tg=9)

    xp2 = _restack(y1, bsz=bsz, cout=16, ho=58, wwide=60, wo=58, hcp=37)
    y2 = _conv_stage(xp2, _pad_lanes(m2, 1024), w2, b2, g2, be2,
                     ac=2, mc=2, wc=29, ho=28, wo=28, cout=32, cin2=32,
                     bsz=bsz, ck=256, nck=4, tg=8)

    xp3 = _restack(y2, bsz=bsz, cout=32, ho=28, wwide=29, wo=28, hcp=20)
    y3 = _conv_stage(xp3, _pad_lanes(m3, 256), w3, b3, g3, be3,
                     ac=2, mc=2, wc=14, ho=13, wo=13, cout=64, cin2=64,
                     bsz=bsz, ck=256, nck=1, tg=4)

    xp4 = _restack(y3, bsz=bsz, cout=64, ho=13, wwide=14, wo=13, hcp=20)
    out = _final_stage(xp4, _pad_lanes(m4, 128), w4, b4, g4, be4, wl, bl,
                       bsz=bsz, tg=4)
    return out[:n]


# speedup vs baseline: 19.3455x; 1.8948x over previous
import functools

import jax
import jax.numpy as jnp
from jax.experimental import pallas as pl
from jax.experimental.pallas import tpu as pltpu

_LN_EPS = 1e-5
_VMEM_LIMIT = 64 * 1024 * 1024


def _conv_ln_body(mask_ref, xp_ref, w_ref, b_ref, g_ref, be_ref, out_ref, y_ref,
                  *scr, bsz, ac, mc, wc, ho, wo, cout, cin2, ck, nck, tg, in5d):
    rows = ho * wc
    inv_cnt = 1.0 / float(cout * ho * wo)
    taps = [(p, a, m) for p in range(2) for a in range(ac) for m in range(mc)]
    ntap = len(taps)
    bias = b_ref[...]
    for s in range(bsz):
        if in5d is not None:
            hcp, wcin = in5d
            xpf = scr[0]
            for p in range(2):
                for hr in range(hcp):
                    xpf[p, :, pl.ds(hr * wcin, wcin)] = xp_ref[s, p, :, hr, :]
        s1 = jnp.zeros((cout, ck), jnp.float32)
        s2 = jnp.zeros((cout, ck), jnp.float32)
        for k in range(nck):
            base = k * ck
            yc = None
            for g0 in range(0, ntap, tg):
                grp = taps[g0:g0 + tg]
                if in5d is not None:
                    rhs = jnp.concatenate(
                        [xpf[p, :, pl.ds(a * wc + m + base, ck)]
                         for (p, a, m) in grp], axis=0)
                else:
                    rhs = jnp.concatenate(
                        [xp_ref[s, p, :, pl.ds(a * wc + m + base, ck)]
                         for (p, a, m) in grp], axis=0)
                d = jnp.dot(w_ref[:, g0 * cin2:(g0 + len(grp)) * cin2], rhs,
                            preferred_element_type=jnp.float32)
                yc = d if yc is None else yc + d
            yc = jnp.maximum(yc + bias, 0.0)
            y_ref[:, base:base + ck] = yc
            t = yc * mask_ref[:, base:base + ck]
            s1 = s1 + t
            s2 = s2 + t * yc
        mu = jnp.sum(s1, keepdims=True) * inv_cnt
        ex2 = jnp.sum(s2, keepdims=True) * inv_cnt
        rstd = jax.lax.rsqrt(ex2 - mu * mu + _LN_EPS)
        y = y_ref[:, :rows]
        rg = g_ref[...].astype(jnp.float32) * rstd
        out_ref[s] = ((y - mu) * rg
                      + be_ref[...].astype(jnp.float32)).astype(out_ref.dtype)


def _final_body(mask_ref, xp_ref, w_ref, b_ref, g_ref, be_ref, wl_ref, bl_ref,
                out_ref, pool_ref, *, bsz, tg):
    ac, mc, wc, ho, wo, cout, cin2, ck = 2, 2, 7, 6, 6, 128, 128, 128
    inv_cnt = 1.0 / float(cout * ho * wo)
    inv_pool = 1.0 / float(ho * wo)
    taps = [(p, a, m) for p in range(2) for a in range(ac) for m in range(mc)]
    mask = mask_ref[...]
    bias = b_ref[...]
    for s in range(bsz):
        yc = None
        for g0 in range(0, len(taps), tg):
            grp = taps[g0:g0 + tg]
            rhs = jnp.concatenate(
                [xp_ref[s, p, :, pl.ds(a * wc + m, ck)] for (p, a, m) in grp],
                axis=0)
            d = jnp.dot(w_ref[:, g0 * cin2:(g0 + len(grp)) * cin2], rhs,
                        preferred_element_type=jnp.float32)
            yc = d if yc is None else yc + d
        yc = jnp.maximum(yc + bias, 0.0)
        t = yc * mask
        mu = jnp.sum(t, keepdims=True) * inv_cnt
        ex2 = jnp.sum(t * yc, keepdims=True) * inv_cnt
        rstd = jax.lax.rsqrt(ex2 - mu * mu + _LN_EPS)
        rg = g_ref[...].astype(jnp.float32) * rstd
        yn = (yc[:, :ho * wc] - mu) * rg + be_ref[...].astype(jnp.float32)
        pool_ref[:, s:s + 1] = jnp.sum(yn * mask[:, :ho * wc], axis=1,
                                       keepdims=True) * inv_pool
    res = (jnp.dot(wl_ref[...], pool_ref[...],
                   preferred_element_type=jnp.float32) + bl_ref[...])
    out_ref[...] = jnp.transpose(res, (1, 0))


def _conv_stage(xp, mask, w2d, bias, gamma, beta, *, ac, mc, wc, ho, wo, cout,
                cin2, bsz, ck, nck, tg, in5d=None):
    rows = ho * wc
    kp = 2 * ac * mc * cin2
    n = xp.shape[0]
    body = functools.partial(_conv_ln_body, bsz=bsz, ac=ac, mc=mc, wc=wc, ho=ho,
                             wo=wo, cout=cout, cin2=cin2, ck=ck, nck=nck, tg=tg,
                             in5d=in5d)
    flops = 2 * n * cout * kp * rows
    bytes_acc = n * (2 * cin2 * 4000 + cout * rows) * 2
    scratch = [pltpu.VMEM((cout, nck * ck), jnp.float32)]
    if in5d is not None:
        hcp, wcin = in5d
        xp_spec = pl.BlockSpec((bsz, 2, cin2, hcp, wcin),
                               lambda i: (i, 0, 0, 0, 0))
        scratch.append(pltpu.VMEM((2, cin2, hcp * wcin), jnp.bfloat16))
    else:
        r = xp.shape[3]
        xp_spec = pl.BlockSpec((bsz, 2, cin2, r), lambda i: (i, 0, 0, 0))
    return pl.pallas_call(
        body,
        out_shape=jax.ShapeDtypeStruct((n, cout, rows), jnp.bfloat16),
        grid=(n // bsz,),
        in_specs=[
            pl.BlockSpec((1, nck * ck), lambda i: (0, 0)),
            xp_spec,
            pl.BlockSpec((cout, kp), lambda i: (0, 0)),
            pl.BlockSpec((cout, 1), lambda i: (0, 0)),
            pl.BlockSpec((cout, rows), lambda i: (0, 0)),
            pl.BlockSpec((cout, rows), lambda i: (0, 0)),
        ],
        out_specs=pl.BlockSpec((bsz, cout, rows), lambda i: (i, 0, 0)),
        scratch_shapes=scratch,
        compiler_params=pltpu.CompilerParams(
            dimension_semantics=("parallel",), vmem_limit_bytes=_VMEM_LIMIT),
        cost_estimate=pl.CostEstimate(flops=int(flops), transcendentals=int(n),
                                      bytes_accessed=int(bytes_acc)),
    )(mask, xp, w2d, bias, gamma, beta)


def _final_stage(xp, mask, w2d, bias, gamma, beta, wl, bl, *, bsz, tg):
    n = xp.shape[0]
    r = xp.shape[3]
    kp = 8 * 128
    flops = 2 * n * (128 * kp * 42 + 27 * 128)
    bytes_acc = n * (2 * 128 * r + 27) * 2
    return pl.pallas_call(
        functools.partial(_final_body, bsz=bsz, tg=tg),
        out_shape=jax.ShapeDtypeStruct((n, 27), jnp.float32),
        grid=(n // bsz,),
        in_specs=[
            pl.BlockSpec((1, 128), lambda i: (0, 0)),
            pl.BlockSpec((bsz, 2, 128, r), lambda i: (i, 0, 0, 0)),
            pl.BlockSpec((128, kp), lambda i: (0, 0)),
            pl.BlockSpec((128, 1), lambda i: (0, 0)),
            pl.BlockSpec((128, 42), lambda i: (0, 0)),
            pl.BlockSpec((128, 42), lambda i: (0, 0)),
            pl.BlockSpec((27, 128), lambda i: (0, 0)),
            pl.BlockSpec((27, 1), lambda i: (0, 0)),
        ],
        out_specs=pl.BlockSpec((bsz, 27), lambda i: (i, 0)),
        scratch_shapes=[pltpu.VMEM((128, bsz), jnp.float32)],
        compiler_params=pltpu.CompilerParams(
            dimension_semantics=("parallel",), vmem_limit_bytes=_VMEM_LIMIT),
        cost_estimate=pl.CostEstimate(flops=int(flops), transcendentals=int(n),
                                      bytes_accessed=int(bytes_acc)),
    )(mask, xp, w2d, bias, gamma, beta, wl, bl)


def _deinterleave_idx(shape, dim, npair):
    j = jax.lax.broadcasted_iota(jnp.int32, shape, dim)
    return jnp.where(j < npair, 2 * j, 2 * (j - npair) + 1)


def _input_restack_body(x_ref, out_ref, *, bsz):
    out_ref[...] = jnp.zeros(out_ref.shape, out_ref.dtype)
    hidx = _deinterleave_idx((840, 120), 1, 60)
    for s in range(bsz):
        t = jnp.transpose(x_ref[s], (1, 0))
        g = jnp.take_along_axis(t, hidx, axis=1)
        g = g.reshape(60, 14, 120)
        for p in range(2):
            for q in range(2):
                for c in range(7):
                    v = g[:, 7 * q + c, 60 * p:60 * (p + 1)]
                    out_ref[s, p, q * 8 + c, :60, :] = (
                        jnp.transpose(v, (1, 0)).astype(out_ref.dtype))


def _restack_flat_body(y_ref, out_ref, *, bsz, cout, ho, wwide, wo, hc, wc,
                       rout):
    out_ref[...] = jnp.zeros(out_ref.shape, out_ref.dtype)
    he = ho // 2
    for s in range(bsz):
        yt = jnp.transpose(y_ref[s].astype(jnp.float32), (1, 0))
        t3 = yt.reshape(ho, wwide, cout)
        for p in range(2):
            hpart = t3[:2 * he].reshape(he, 2, wwide, cout)[:, p]
            for q in range(2):
                wv = min((wo - q + 1) // 2, wc)
                piece = hpart[:, :2 * wc, :].reshape(he, wc, 2, cout)[:, :, q]
                if wv < wc:
                    piece = jnp.concatenate(
                        [piece[:, :wv], jnp.zeros((he, wc - wv, cout),
                                                  piece.dtype)], axis=1)
                z = jnp.transpose(piece.reshape(he * wc, cout), (1, 0))
                out_ref[s, p, q * cout:(q + 1) * cout, :he * wc] = (
                    z.astype(out_ref.dtype))
                if p == 0 and ho % 2 == 1:
                    lr = t3[ho - 1, :2 * wc, :].reshape(wc, 2, cout)[:, q]
                    if wv < wc:
                        lr = jnp.concatenate(
                            [lr[:wv], jnp.zeros((wc - wv, cout), lr.dtype)],
                            axis=0)
                    out_ref[s, p, q * cout:(q + 1) * cout,
                            pl.ds(he * wc, wc)] = (
                        jnp.transpose(lr, (1, 0)).astype(out_ref.dtype))


def _input_restack(x, *, bsz):
    n = x.shape[0]
    return pl.pallas_call(
        functools.partial(_input_restack_body, bsz=bsz),
        out_shape=jax.ShapeDtypeStruct((n, 2, 16, 62, 60), jnp.bfloat16),
        grid=(n // bsz,),
        in_specs=[pl.BlockSpec((bsz, 120, 840), lambda i: (i, 0, 0))],
        out_specs=pl.BlockSpec((bsz, 2, 16, 62, 60), lambda i: (i, 0, 0, 0, 0)),
        compiler_params=pltpu.CompilerParams(
            dimension_semantics=("parallel",), vmem_limit_bytes=_VMEM_LIMIT),
    )(x)


def _restack(y, *, bsz, cout, ho, wwide, wo, rout):
    n = y.shape[0]
    hc, wc = (ho + 1) // 2, (wo + 1) // 2
    return pl.pallas_call(
        functools.partial(_restack_flat_body, bsz=bsz, cout=cout, ho=ho,
                          wwide=wwide, wo=wo, hc=hc, wc=wc, rout=rout),
        out_shape=jax.ShapeDtypeStruct((n, 2, 2 * cout, rout), jnp.bfloat16),
        grid=(n // bsz,),
        in_specs=[pl.BlockSpec((bsz, cout, ho * wwide), lambda i: (i, 0, 0))],
        out_specs=pl.BlockSpec((bsz, 2, 2 * cout, rout), lambda i: (i, 0, 0, 0)),
        compiler_params=pltpu.CompilerParams(
            dimension_semantics=("parallel",), vmem_limit_bytes=_VMEM_LIMIT),
    )(y)


def _pad_lanes(m, lanes):
    return jnp.pad(m, ((0, 0), (0, lanes - m.shape[1])))


def kernel(x, w1, b1, g1, be1, m1, w2, b2, g2, be2, m2, w3, b3, g3, be3, m3,
           w4, b4, g4, be4, m4, wl, bl):
    n = x.shape[0]
    bsz = 8 if n >= 16 else (4 if n >= 8 else (2 if n >= 4 else 1))
    n_pad = ((n + bsz - 1) // bsz) * bsz
    if n_pad > n:
        x = jnp.pad(x, ((0, n_pad - n), (0, 0), (0, 0), (0, 0)))

    xp1 = _input_restack(x.reshape(n_pad, 120, 840), bsz=bsz)

    y1 = _conv_stage(xp1, _pad_lanes(m1, 3584), w1, b1, g1, be1,
                     ac=3, mc=3, wc=60, ho=58, wo=58, cout=16, cin2=16,
                     bsz=bsz, ck=256, nck=14, tg=9, in5d=(62, 60))

    xp2 = _restack(y1, bsz=bsz, cout=16, ho=58, wwide=60, wo=58, rout=1073)
    y2 = _conv_stage(xp2, _pad_lanes(m2, 1024), w2, b2, g2, be2,
                     ac=2, mc=2, wc=29, ho=28, wo=28, cout=32, cin2=32,
                     bsz=bsz, ck=256, nck=4, tg=8)

    xp3 = _restack(y2, bsz=bsz, cout=32, ho=28, wwide=29, wo=28, rout=280)
    y3 = _conv_stage(xp3, _pad_lanes(m3, 256), w3, b3, g3, be3,
                     ac=2, mc=2, wc=14, ho=13, wo=13, cout=64, cin2=64,
                     bsz=bsz, ck=256, nck=1, tg=4)

    xp4 = _restack(y3, bsz=bsz, cout=64, ho=13, wwide=14, wo=13, rout=140)
    out = _final_stage(xp4, _pad_lanes(m4, 128), w4, b4, g4, be4, wl, bl,
                       bsz=bsz, tg=4)
    return out[:n]
